# baseline (speedup 1.0000x reference)
import functools

import jax
import jax.numpy as jnp
from jax import lax
from jax.experimental import pallas as pl
from jax.experimental.pallas import tpu as pltpu
from jax.experimental.pallas import tpu_sc as plsc

NC = 2
NS = 16
L = 16
NW = NC * NS
IDX_CHUNK = 128
OUT_BLK = 184
N_OBLKS = 17
W_ROWS = OUT_BLK * N_OBLKS


def _sc_decode(table, idx3, latent, slope1):
    V, D = table.shape
    n_chunks = idx3.shape[1]
    b_per_w = n_chunks * IDX_CHUNK
    B = NW * b_per_w
    sg_total = V // 8
    mesh = plsc.VectorSubcoreMesh(core_axis_name="c", subcore_axis_name="s")

    @functools.partial(
        pl.kernel,
        mesh=mesh,
        compiler_params=pltpu.CompilerParams(needs_layout_passes=False),
        out_type=(
            jax.ShapeDtypeStruct((B, D), jnp.float32),
            jax.ShapeDtypeStruct((V, D), jnp.float32),
        ),
        scratch_types=[
            pltpu.VMEM((n_chunks, IDX_CHUNK), jnp.int32),
            pltpu.VMEM((b_per_w, D), jnp.float32),
            pltpu.VMEM((D,), jnp.float32),
            pltpu.VMEM((W_ROWS,), jnp.float32),
            pltpu.VMEM((2, OUT_BLK, D), jnp.float32),
            pltpu.SemaphoreType.DMA,
            pltpu.SemaphoreType.DMA,
            pltpu.SemaphoreType.DMA,
        ],
    )
    def k(table_hbm, idx_hbm, latent_hbm, slope_hbm, height_hbm, overall_hbm,
          idx_v, rows_v, lat_v, slope_v, obuf_v, gsem, wsem0, wsem1):
        wid = lax.axis_index("s") * NC + lax.axis_index("c")
        hbase = wid * b_per_w
        obase = pl.multiple_of(((wid * sg_total) // NW) * 8, 8)
        pltpu.sync_copy(latent_hbm, lat_v)
        pltpu.sync_copy(idx_hbm.at[wid], idx_v)
        pltpu.sync_copy(slope_hbm.at[pl.ds(obase, W_ROWS)], slope_v)
        gathers = [
            pltpu.async_copy(
                table_hbm.at[idx_v.at[t]],
                rows_v.at[pl.ds(t * IDX_CHUNK, IDX_CHUNK)],
                gsem,
            )
            for t in range(n_chunks)
        ]
        lat = [lat_v[pl.ds(j * L, L)] for j in range(D // L)]

        wsems = (wsem0, wsem1)
        pending = [None, None]
        for blk in range(N_OBLKS):
            par = blk % 2
            if pending[par] is not None:
                pending[par].wait()

            @plsc.parallel_loop(0, OUT_BLK, unroll=8)
            def obody(i, _blk=blk, _par=par):
                bi = jnp.broadcast_to(_blk * OUT_BLK + i, (L,))
                s = plsc.load_gather(slope_v, [bi])
                for j in range(D // L):
                    obuf_v[_par, i, pl.ds(j * L, L)] = s * lat[j]

            pending[par] = pltpu.async_copy(
                obuf_v.at[par],
                overall_hbm.at[pl.ds(obase + blk * OUT_BLK, OUT_BLK)],
                wsems[par],
            )

        for g in gathers:
            g.wait()

        @plsc.parallel_loop(0, b_per_w, unroll=8)
        def sbody(i):
            for j in range(D // L):
                rows_v[i, pl.ds(j * L, L)] = rows_v[i, pl.ds(j * L, L)] * lat[j]

        pltpu.sync_copy(rows_v, height_hbm.at[pl.ds(hbase, b_per_w)])
        for p in pending:
            if p is not None:
                p.wait()

    return k(table, idx3, latent, slope1)


def kernel(latent, genes_oi, height_slope_weight, overall_slope_weight):
    B = genes_oi.shape[0]
    V, _, D = height_slope_weight.shape
    n_chunks = B // (NW * IDX_CHUNK)
    gathered, overall = _sc_decode(
        height_slope_weight.reshape(V, D),
        genes_oi.reshape(NW, n_chunks, IDX_CHUNK),
        latent,
        overall_slope_weight.reshape(V),
    )
    return (gathered.reshape(B, 1, D), overall)


# speedup vs baseline: 1.0580x; 1.0580x over previous
import functools

import jax
import jax.numpy as jnp
from jax import lax
from jax.experimental import pallas as pl
from jax.experimental.pallas import tpu as pltpu
from jax.experimental.pallas import tpu_sc as plsc

NC = 2
NS = 16
L = 16
NW = NC * NS
IDX_CHUNK = 128
OUT_BLK = 184
N_OBLKS = 17
W_ROWS = OUT_BLK * N_OBLKS


def _sc_decode(table, idx3, latent, slope1):
    V, D = table.shape
    n_chunks = idx3.shape[1]
    b_per_w = n_chunks * IDX_CHUNK
    B = NW * b_per_w
    sg_total = V // 8
    mesh = plsc.VectorSubcoreMesh(core_axis_name="c", subcore_axis_name="s")

    @functools.partial(
        pl.kernel,
        mesh=mesh,
        compiler_params=pltpu.CompilerParams(needs_layout_passes=False),
        out_type=(
            jax.ShapeDtypeStruct((B, D), jnp.float32),
            jax.ShapeDtypeStruct((V, D), jnp.float32),
        ),
        scratch_types=[
            pltpu.VMEM((n_chunks, IDX_CHUNK), jnp.int32),
            pltpu.VMEM((b_per_w, D), jnp.float32),
            pltpu.VMEM((D,), jnp.float32),
            pltpu.VMEM((W_ROWS,), jnp.float32),
            pltpu.VMEM((2, OUT_BLK, D), jnp.float32),
            pltpu.SemaphoreType.DMA,
            pltpu.SemaphoreType.DMA,
            pltpu.SemaphoreType.DMA,
        ],
    )
    def k(table_hbm, idx_hbm, latent_hbm, slope_hbm, height_hbm, overall_hbm,
          idx_v, rows_v, lat_v, slope_v, obuf_v, gsem, wsem0, wsem1):
        wid = lax.axis_index("s") * NC + lax.axis_index("c")
        hbase = wid * b_per_w
        obase = pl.multiple_of(((wid * sg_total) // NW) * 8, 8)
        pltpu.sync_copy(latent_hbm, lat_v)
        pltpu.sync_copy(idx_hbm.at[wid], idx_v)
        pltpu.sync_copy(slope_hbm.at[pl.ds(obase, W_ROWS)], slope_v)
        gathers = [
            pltpu.async_copy(
                table_hbm.at[idx_v.at[t]],
                rows_v.at[pl.ds(t * IDX_CHUNK, IDX_CHUNK)],
                gsem,
            )
            for t in range(n_chunks)
        ]
        lat = [lat_v[pl.ds(j * L, L)] for j in range(D // L)]

        wsems = (wsem0, wsem1)
        pending = [None, None]
        for blk in range(N_OBLKS):
            par = blk % 2
            if pending[par] is not None:
                pending[par].wait()

            @plsc.parallel_loop(0, OUT_BLK, unroll=4)
            def obody(i, _blk=blk, _par=par):
                bi = jnp.broadcast_to(_blk * OUT_BLK + i, (L,))
                s = plsc.load_gather(slope_v, [bi])
                for j in range(D // L):
                    obuf_v[_par, i, pl.ds(j * L, L)] = s * lat[j]

            pending[par] = pltpu.async_copy(
                obuf_v.at[par],
                overall_hbm.at[pl.ds(obase + blk * OUT_BLK, OUT_BLK)],
                wsems[par],
            )

        for g in gathers:
            g.wait()

        @plsc.parallel_loop(0, b_per_w, unroll=4)
        def sbody(i):
            for j in range(D // L):
                rows_v[i, pl.ds(j * L, L)] = rows_v[i, pl.ds(j * L, L)] * lat[j]

        pltpu.sync_copy(rows_v, height_hbm.at[pl.ds(hbase, b_per_w)])
        for p in pending:
            if p is not None:
                p.wait()

    return k(table, idx3, latent, slope1)


def kernel(latent, genes_oi, height_slope_weight, overall_slope_weight):
    B = genes_oi.shape[0]
    V, _, D = height_slope_weight.shape
    n_chunks = B // (NW * IDX_CHUNK)
    gathered, overall = _sc_decode(
        height_slope_weight.reshape(V, D),
        genes_oi.reshape(NW, n_chunks, IDX_CHUNK),
        latent,
        overall_slope_weight.reshape(V),
    )
    return (gathered.reshape(B, 1, D), overall)


# speedup vs baseline: 1.0878x; 1.0281x over previous
import functools

import jax
import jax.numpy as jnp
from jax import lax
from jax.experimental import pallas as pl
from jax.experimental.pallas import tpu as pltpu
from jax.experimental.pallas import tpu_sc as plsc

NC = 2
NS = 16
L = 16
NW = NC * NS
IDX_CHUNK = 128
OUT_BLK = 184
N_OBLKS = 17
W_ROWS = OUT_BLK * N_OBLKS


def _sc_decode(table, idx3, latent, slope1):
    V, D = table.shape
    n_chunks = idx3.shape[1]
    b_per_w = n_chunks * IDX_CHUNK
    B = NW * b_per_w
    sg_total = V // 8
    mesh = plsc.VectorSubcoreMesh(core_axis_name="c", subcore_axis_name="s")

    @functools.partial(
        pl.kernel,
        mesh=mesh,
        compiler_params=pltpu.CompilerParams(needs_layout_passes=False),
        out_type=(
            jax.ShapeDtypeStruct((B, D), jnp.float32),
            jax.ShapeDtypeStruct((V, D), jnp.float32),
        ),
        scratch_types=[
            pltpu.VMEM((n_chunks, IDX_CHUNK), jnp.int32),
            pltpu.VMEM((b_per_w, D), jnp.float32),
            pltpu.VMEM((D,), jnp.float32),
            pltpu.VMEM((W_ROWS,), jnp.float32),
            pltpu.VMEM((2, OUT_BLK, D), jnp.float32),
            pltpu.SemaphoreType.DMA,
            pltpu.SemaphoreType.DMA,
            pltpu.SemaphoreType.DMA,
            pltpu.SemaphoreType.DMA,
        ],
    )
    def k(table_hbm, idx_hbm, latent_hbm, slope_hbm, height_hbm, overall_hbm,
          idx_v, rows_v, lat_v, slope_v, obuf_v, gsem, wsem0, wsem1, ssem):
        wid = lax.axis_index("s") * NC + lax.axis_index("c")
        hbase = wid * b_per_w
        obase = pl.multiple_of(((wid * sg_total) // NW) * 8, 8)
        pltpu.sync_copy(idx_hbm.at[wid], idx_v)
        gathers = [
            pltpu.async_copy(
                table_hbm.at[idx_v.at[t]],
                rows_v.at[pl.ds(t * IDX_CHUNK, IDX_CHUNK)],
                gsem,
            )
            for t in range(n_chunks)
        ]
        lcopy = pltpu.async_copy(latent_hbm, lat_v, ssem)
        scopy = pltpu.async_copy(slope_hbm.at[pl.ds(obase, W_ROWS)], slope_v,
                                 ssem)
        lcopy.wait()
        scopy.wait()
        lat = [lat_v[pl.ds(j * L, L)] for j in range(D // L)]

        wsems = (wsem0, wsem1)
        pending = [None, None]
        for blk in range(N_OBLKS):
            par = blk % 2
            if pending[par] is not None:
                pending[par].wait()

            @plsc.parallel_loop(0, OUT_BLK, unroll=4)
            def obody(i, _blk=blk, _par=par):
                bi = jnp.broadcast_to(_blk * OUT_BLK + i, (L,))
                s = plsc.load_gather(slope_v, [bi])
                for j in range(D // L):
                    obuf_v[_par, i, pl.ds(j * L, L)] = s * lat[j]

            pending[par] = pltpu.async_copy(
                obuf_v.at[par],
                overall_hbm.at[pl.ds(obase + blk * OUT_BLK, OUT_BLK)],
                wsems[par],
            )

        for g in gathers:
            g.wait()

        @plsc.parallel_loop(0, b_per_w, unroll=4)
        def sbody(i):
            for j in range(D // L):
                rows_v[i, pl.ds(j * L, L)] = rows_v[i, pl.ds(j * L, L)] * lat[j]

        pltpu.sync_copy(rows_v, height_hbm.at[pl.ds(hbase, b_per_w)])
        for p in pending:
            if p is not None:
                p.wait()

    return k(table, idx3, latent, slope1)


def kernel(latent, genes_oi, height_slope_weight, overall_slope_weight):
    B = genes_oi.shape[0]
    V, _, D = height_slope_weight.shape
    n_chunks = B // (NW * IDX_CHUNK)
    gathered, overall = _sc_decode(
        height_slope_weight.reshape(V, D),
        genes_oi.reshape(NW, n_chunks, IDX_CHUNK),
        latent,
        overall_slope_weight.reshape(V),
    )
    return (gathered.reshape(B, 1, D), overall)
